# baseline (speedup 1.0000x reference)
import functools

import jax
import jax.numpy as jnp
from jax import lax
from jax.experimental import pallas as pl
from jax.experimental.pallas import tpu as pltpu
from jax.experimental.pallas import tpu_sc as plsc

_N = 50000
_M = 16
_F = 128
_K = 4
_NH = 16
_D = _K * _NH
_ROUTIT = 6
_B = 2000
_NC = 2
_NS = 16
_NW = _NC * _NS
_CH = 128


def _reduce_mat(g, d):
    i = lax.broadcasted_iota(jnp.int32, (d, d // g), 0)
    c = lax.broadcasted_iota(jnp.int32, (d, d // g), 1)
    return (i // g == c).astype(jnp.float32)


def _expand_mat(g, d):
    c = lax.broadcasted_iota(jnp.int32, (d // g, d), 0)
    i = lax.broadcasted_iota(jnp.int32, (d // g, d), 1)
    return (i // g == c).astype(jnp.float32)


def _dot(a, b):
    return jnp.dot(a, b, preferred_element_type=jnp.float32)


def _cap_normalize(u, r4, e4):
    nsq = _dot(u * u, r4)
    inv = 1.0 / jnp.maximum(jnp.sqrt(nsq), 1e-12)
    return u * _dot(inv, e4)


def _pca_body(x_ref, w_ref, b_ref, o_ref, ob_ref):
    h = jnp.maximum(_dot(x_ref[...], w_ref[...]) + b_ref[...], 0.0)
    r4 = _reduce_mat(_NH, _D)
    e4 = _expand_mat(_NH, _D)
    hn = _cap_normalize(h, r4, e4)
    o_ref[...] = hn
    ob_ref[...] = hn.astype(jnp.bfloat16)


def _pca(x, pca_W, pca_b):
    grid = (_N // _B,)
    return pl.pallas_call(
        _pca_body,
        grid=grid,
        in_specs=[
            pl.BlockSpec((_B, _F), lambda i: (i, 0)),
            pl.BlockSpec((_F, _D), lambda i: (0, 0)),
            pl.BlockSpec((1, _D), lambda i: (0, 0)),
        ],
        out_specs=[
            pl.BlockSpec((_B, _D), lambda i: (i, 0)),
            pl.BlockSpec((_B, _D), lambda i: (i, 0)),
        ],
        out_shape=[
            jax.ShapeDtypeStruct((_N, _D), jnp.float32),
            jax.ShapeDtypeStruct((_N, _D), jnp.bfloat16),
        ],
    )(x, pca_W, pca_b.reshape(1, _D))


_PER_W = 198
_RPAD = _PER_W * _NW * _CH


_DW = _D // 2


def _gather_sc(hn_packed, idx_pad):
    mesh = plsc.VectorSubcoreMesh(core_axis_name="c", subcore_axis_name="s")

    @functools.partial(
        pl.kernel,
        mesh=mesh,
        out_type=jax.ShapeDtypeStruct((_RPAD, _DW), jnp.int32),
        scratch_types=[
            pltpu.VMEM((_PER_W * _CH,), jnp.int32),
            pltpu.VMEM((_CH, _DW), jnp.int32),
            pltpu.VMEM((_CH, _DW), jnp.int32),
            pltpu.VMEM((_CH, _DW), jnp.int32),
            pltpu.SemaphoreType.DMA,
            pltpu.SemaphoreType.DMA,
            pltpu.SemaphoreType.DMA,
            pltpu.SemaphoreType.DMA,
            pltpu.SemaphoreType.DMA,
            pltpu.SemaphoreType.DMA,
        ],
        compiler_params=pltpu.CompilerParams(use_tc_tiling_on_sc=False),
    )
    def k(hn_hbm, idx_hbm, out_hbm, idx_v, r0, r1, r2,
          g0, g1, g2, w0, w1, w2):
        wid = lax.axis_index("s") * _NC + lax.axis_index("c")
        base = wid * _PER_W
        rbufs, gsems, wsems = (r0, r1, r2), (g0, g1, g2), (w0, w1, w2)

        pltpu.sync_copy(idx_hbm.at[pl.ds(base * _CH, _PER_W * _CH)], idx_v)

        def gath_start(j, b):
            pltpu.async_copy(
                hn_hbm.at[idx_v.at[pl.ds(j * _CH, _CH)]], rbufs[b], gsems[b])

        def gath_wait(b):
            pltpu.make_async_copy(
                hn_hbm.at[idx_v.at[pl.ds(0, _CH)]], rbufs[b], gsems[b]).wait()

        def wback_start(j, b):
            pltpu.async_copy(
                rbufs[b], out_hbm.at[pl.ds((base + j) * _CH, _CH)], wsems[b])

        def wback_wait(b):
            pltpu.make_async_copy(
                rbufs[b], out_hbm.at[pl.ds(base * _CH, _CH)], wsems[b]).wait()

        gath_start(0, 0)
        gath_start(1, 1)

        def body(p, carry):
            for b in range(3):
                j = 3 * p + b
                b2 = (b + 2) % 3
                gath_wait(b)
                wback_start(j, b)
                ok = jnp.logical_and(j >= 1, j < _PER_W - 2)

                @pl.when(ok)
                def _():
                    wback_wait(b2)

                @pl.when(j < _PER_W - 2)
                def _():
                    gath_start(j + 2, b2)

            return carry

        lax.fori_loop(0, _PER_W // 3, body, 0)
        for b in range(3):
            wback_wait(b)

    return k(hn_packed, idx_pad)


_W = _M * _D


def _np_consts():
    import numpy as np
    i = np.arange(_W)
    m_i, k_i = i // _D, (i % _D) // _NH
    c = np.arange(_M * _K)
    r_all = (m_i[:, None] * _K + k_i[:, None] == c[None, :]).astype(np.float32)
    g_all = (c[:, None] // _K == c[None, :] // _K).astype(np.float32)
    r4 = (np.arange(_D)[:, None] // _NH == np.arange(_K)[None, :]).astype(
        np.float32)
    return (jnp.asarray(r_all, dtype=jnp.bfloat16),
            jnp.asarray(r_all.T, dtype=jnp.bfloat16), jnp.asarray(g_all),
            jnp.asarray(r4), jnp.asarray(r4.T))


def _fold16(w):
    acc = None
    for m in range(0, _M, 2):
        wp = (w[:, m * _D:(m + 1) * _D]
              + w[:, (m + 1) * _D:(m + 2) * _D]).astype(jnp.float32)
        acc = wp if acc is None else acc + wp
    return acc


def _route_body(z_ref, xn_ref, rall_ref, eall_ref, gall_ref, r4_ref, e4_ref,
                o_ref, ob_ref, *, final):
    r_all, e_all, g_all = rall_ref[...], eall_ref[...], gall_ref[...]
    r4, e4 = r4_ref[...], e4_ref[...]

    _H = _B // 2
    zs = [z_ref[0:_H, :], z_ref[_H:_B, :]]
    xns = [xn_ref[0:_H, :], xn_ref[_H:_B, :]]
    us = [None, None]

    for h in range(2):
        us[h] = _cap_normalize((1.0 / _K) * _fold16(zs[h]) + xns[h], r4, e4)

    for t in range(1, _ROUTIT):
        for h in range(2):
            z, xn, u = zs[h], xns[h], us[h]
            uw = jnp.concatenate([u] * _M, axis=1).astype(jnp.bfloat16)
            p = _dot(z * uw, r_all)
            e = jnp.exp(p)
            s = _dot(e, g_all)
            pb = _dot((e / s).astype(jnp.bfloat16),
                      e_all).astype(jnp.bfloat16)
            u = _fold16(z * pb) + xn
            if t < _ROUTIT - 1:
                u = _cap_normalize(u, r4, e4)
            us[h] = u

    for h in range(2):
        sl = slice(0, _H) if h == 0 else slice(_H, _B)
        if final:
            o_ref[sl, :] = us[h]
            ob_ref[sl, :] = us[h].astype(jnp.bfloat16)
        else:
            hn = _cap_normalize(jnp.maximum(us[h], 0.0), r4, e4)
            o_ref[sl, :] = hn
            ob_ref[sl, :] = hn.astype(jnp.bfloat16)


def _route(z2d, xn, final):
    grid = (_N // _B,)
    full = lambda shape: pl.BlockSpec(shape, lambda i: tuple(0 for _ in shape))
    return pl.pallas_call(
        functools.partial(_route_body, final=final),
        grid=grid,
        in_specs=[
            pl.BlockSpec((_B, _W), lambda i: (i, 0)),
            pl.BlockSpec((_B, _D), lambda i: (i, 0)),
            full((_W, _M * _K)),
            full((_M * _K, _W)),
            full((_M * _K, _M * _K)),
            full((_D, _K)),
            full((_K, _D)),
        ],
        out_specs=[
            pl.BlockSpec((_B, _D), lambda i: (i, 0)),
            pl.BlockSpec((_B, _D), lambda i: (i, 0)),
        ],
        out_shape=[
            jax.ShapeDtypeStruct((_N, _D), jnp.float32),
            jax.ShapeDtypeStruct((_N, _D), jnp.bfloat16),
        ],
    )(z2d, xn, *_np_consts())


def _head_body(u_ref, awf_ref, attpW_ref, attpb_ref, predW_ref, predb_ref,
               o1_ref, o2_ref, o3_ref, o4_ref):
    u = u_ref[...]
    r4 = _reduce_mat(_NH, _D)
    e4 = _expand_mat(_NH, _D)
    scores = _dot(u * awf_ref[...], r4)
    m = jnp.max(scores, axis=1, keepdims=True)
    e = jnp.exp(scores - m)
    att = e / jnp.sum(e, axis=1, keepdims=True)
    h_att = u * _dot(att, e4)

    ar = _dot(att, attpW_ref[...]) + attpb_ref[...]
    arm = jnp.max(ar, axis=1, keepdims=True)
    ars = ar - arm
    o3 = ars - jnp.log(jnp.sum(jnp.exp(ars), axis=1, keepdims=True))

    logits = _dot(h_att, predW_ref[...]) + predb_ref[...]
    lm = jnp.max(logits, axis=1, keepdims=True)
    ls = logits - lm
    o1 = ls - jnp.log(jnp.sum(jnp.exp(ls), axis=1, keepdims=True))

    o1_ref[...] = o1
    o2_ref[...] = att
    o3_ref[...] = o3
    o4_ref[...] = h_att


def _head(u, att_w, attp_W, attp_b, pred_W, pred_b):
    nclass = pred_W.shape[1]
    awf = att_w.reshape(1, _D)
    predWt = jnp.concatenate([pred_W] * _K, axis=0)
    grid = (_N // _B,)
    full = lambda shape: pl.BlockSpec(shape, lambda i: tuple(0 for _ in shape))
    return pl.pallas_call(
        _head_body,
        grid=grid,
        in_specs=[
            pl.BlockSpec((_B, _D), lambda i: (i, 0)),
            full((1, _D)),
            full((_K, nclass)),
            full((1, nclass)),
            full((_D, nclass)),
            full((1, nclass)),
        ],
        out_specs=[
            pl.BlockSpec((_B, nclass), lambda i: (i, 0)),
            pl.BlockSpec((_B, _K), lambda i: (i, 0)),
            pl.BlockSpec((_B, nclass), lambda i: (i, 0)),
            pl.BlockSpec((_B, _D), lambda i: (i, 0)),
        ],
        out_shape=[
            jax.ShapeDtypeStruct((_N, nclass), jnp.float32),
            jax.ShapeDtypeStruct((_N, _K), jnp.float32),
            jax.ShapeDtypeStruct((_N, nclass), jnp.float32),
            jax.ShapeDtypeStruct((_N, _D), jnp.float32),
        ],
    )(u, awf, attp_W, attp_b.reshape(1, nclass), predWt,
      pred_b.reshape(1, nclass))


def kernel(x, nb, pca_W, pca_b, att_w, pred_W, pred_b, attp_W, attp_b):
    n, m = nb.shape
    idx_pad = jnp.concatenate(
        [nb.reshape(-1), jnp.zeros((_RPAD - n * m,), dtype=nb.dtype)])

    hn, hn_bf = _pca(x, pca_W, pca_b)
    for stage in range(3):
        packed = lax.bitcast_convert_type(
            hn_bf.reshape(n, _DW, 2), jnp.int32)
        zi = _gather_sc(packed, idx_pad)
        zb = lax.bitcast_convert_type(zi, jnp.bfloat16)
        z2d = zb.reshape(_RPAD * _D // _W, _W)
        hn, hn_bf = _route(z2d, hn, final=(stage == 2))
    u = hn

    o1, att, o3, h_att = _head(u, att_w, attp_W, attp_b, pred_W, pred_b)
    return (o1, att, o3, h_att, u)


# speedup vs baseline: 20.5711x; 20.5711x over previous
import functools

import jax
import jax.numpy as jnp
from jax import lax
from jax.experimental import pallas as pl
from jax.experimental.pallas import tpu as pltpu
from jax.experimental.pallas import tpu_sc as plsc

_N = 50000
_M = 16
_F = 128
_K = 4
_NH = 16
_D = _K * _NH
_ROUTIT = 6
_B = 2000
_NC = 2
_NS = 16
_NW = _NC * _NS
_CH = 128


def _reduce_mat(g, d):
    i = lax.broadcasted_iota(jnp.int32, (d, d // g), 0)
    c = lax.broadcasted_iota(jnp.int32, (d, d // g), 1)
    return (i // g == c).astype(jnp.float32)


def _expand_mat(g, d):
    c = lax.broadcasted_iota(jnp.int32, (d // g, d), 0)
    i = lax.broadcasted_iota(jnp.int32, (d // g, d), 1)
    return (i // g == c).astype(jnp.float32)


def _dot(a, b):
    return jnp.dot(a, b, preferred_element_type=jnp.float32)


def _pack_bf16(u):
    ub = u.astype(jnp.bfloat16)
    dw = ub.shape[1] // 2
    a = lax.bitcast_convert_type(ub[:, :dw], jnp.uint16).astype(jnp.int32)
    b = lax.bitcast_convert_type(ub[:, dw:], jnp.uint16).astype(jnp.int32)
    return a | (b << 16)


def _unpack_bf16(zi):
    lo = lax.bitcast_convert_type(
        (zi & 0xffff).astype(jnp.uint16), jnp.bfloat16)
    hi = lax.bitcast_convert_type(
        lax.shift_right_logical(zi, 16).astype(jnp.uint16), jnp.bfloat16)
    dw = _D // 2
    parts = []
    for m in range(_M):
        parts.append(lo[:, m * dw:(m + 1) * dw])
        parts.append(hi[:, m * dw:(m + 1) * dw])
    return jnp.concatenate(parts, axis=1)


def _cap_normalize(u, r4, e4):
    nsq = _dot(u * u, r4)
    inv = 1.0 / jnp.maximum(jnp.sqrt(nsq), 1e-12)
    return u * _dot(inv, e4)


def _pca_body(x_ref, w_ref, b_ref, o_ref, ob_ref):
    h = jnp.maximum(_dot(x_ref[...], w_ref[...]) + b_ref[...], 0.0)
    r4 = _reduce_mat(_NH, _D)
    e4 = _expand_mat(_NH, _D)
    hn = _cap_normalize(h, r4, e4)
    o_ref[...] = hn
    ob_ref[...] = _pack_bf16(hn)


def _pca(x, pca_W, pca_b):
    grid = (_N // _B,)
    return pl.pallas_call(
        _pca_body,
        grid=grid,
        in_specs=[
            pl.BlockSpec((_B, _F), lambda i: (i, 0)),
            pl.BlockSpec((_F, _D), lambda i: (0, 0)),
            pl.BlockSpec((1, _D), lambda i: (0, 0)),
        ],
        out_specs=[
            pl.BlockSpec((_B, _D), lambda i: (i, 0)),
            pl.BlockSpec((_B, _DW), lambda i: (i, 0)),
        ],
        out_shape=[
            jax.ShapeDtypeStruct((_N, _D), jnp.float32),
            jax.ShapeDtypeStruct((_N, _DW), jnp.int32),
        ],
    )(x, pca_W, pca_b.reshape(1, _D))


_PER_W = 198
_RPAD = _PER_W * _NW * _CH


_DW = _D // 2


def _gather_sc(hn_packed, idx_pad):
    mesh = plsc.VectorSubcoreMesh(core_axis_name="c", subcore_axis_name="s")

    @functools.partial(
        pl.kernel,
        mesh=mesh,
        out_type=jax.ShapeDtypeStruct((_RPAD, _DW), jnp.int32),
        scratch_types=[
            pltpu.VMEM((_PER_W * _CH,), jnp.int32),
            pltpu.VMEM((_CH, _DW), jnp.int32),
            pltpu.VMEM((_CH, _DW), jnp.int32),
            pltpu.VMEM((_CH, _DW), jnp.int32),
            pltpu.SemaphoreType.DMA,
            pltpu.SemaphoreType.DMA,
            pltpu.SemaphoreType.DMA,
            pltpu.SemaphoreType.DMA,
            pltpu.SemaphoreType.DMA,
            pltpu.SemaphoreType.DMA,
        ],
        compiler_params=pltpu.CompilerParams(use_tc_tiling_on_sc=False),
    )
    def k(hn_hbm, idx_hbm, out_hbm, idx_v, r0, r1, r2,
          g0, g1, g2, w0, w1, w2):
        wid = lax.axis_index("s") * _NC + lax.axis_index("c")
        base = wid * _PER_W
        rbufs, gsems, wsems = (r0, r1, r2), (g0, g1, g2), (w0, w1, w2)

        pltpu.sync_copy(idx_hbm.at[pl.ds(base * _CH, _PER_W * _CH)], idx_v)

        def gath_start(j, b):
            pltpu.async_copy(
                hn_hbm.at[idx_v.at[pl.ds(j * _CH, _CH)]], rbufs[b], gsems[b])

        def gath_wait(b):
            pltpu.make_async_copy(
                hn_hbm.at[idx_v.at[pl.ds(0, _CH)]], rbufs[b], gsems[b]).wait()

        def wback_start(j, b):
            pltpu.async_copy(
                rbufs[b], out_hbm.at[pl.ds((base + j) * _CH, _CH)], wsems[b])

        def wback_wait(b):
            pltpu.make_async_copy(
                rbufs[b], out_hbm.at[pl.ds(base * _CH, _CH)], wsems[b]).wait()

        gath_start(0, 0)
        gath_start(1, 1)

        def body(p, carry):
            for b in range(3):
                j = 3 * p + b
                b2 = (b + 2) % 3
                gath_wait(b)
                wback_start(j, b)
                ok = jnp.logical_and(j >= 1, j < _PER_W - 2)

                @pl.when(ok)
                def _():
                    wback_wait(b2)

                @pl.when(j < _PER_W - 2)
                def _():
                    gath_start(j + 2, b2)

            return carry

        lax.fori_loop(0, _PER_W // 3, body, 0)
        for b in range(3):
            wback_wait(b)

    return k(hn_packed, idx_pad)


_W = _M * _D


def _np_consts():
    import numpy as np
    i = np.arange(_W)
    m_i, k_i = i // _D, (i % _D) // _NH
    c = np.arange(_M * _K)
    r_all = (m_i[:, None] * _K + k_i[:, None] == c[None, :]).astype(np.float32)
    g_all = (c[:, None] // _K == c[None, :] // _K).astype(np.float32)
    r4 = (np.arange(_D)[:, None] // _NH == np.arange(_K)[None, :]).astype(
        np.float32)
    return (jnp.asarray(r_all, dtype=jnp.bfloat16),
            jnp.asarray(r_all.T, dtype=jnp.bfloat16), jnp.asarray(g_all),
            jnp.asarray(r4), jnp.asarray(r4.T))


def _fold16(w):
    acc = None
    for m in range(0, _M, 2):
        wp = (w[:, m * _D:(m + 1) * _D]
              + w[:, (m + 1) * _D:(m + 2) * _D]).astype(jnp.float32)
        acc = wp if acc is None else acc + wp
    return acc


def _route_body(z_ref, xn_ref, rall_ref, eall_ref, gall_ref, r4_ref, e4_ref,
                o_ref, ob_ref, *, final):
    r_all, e_all, g_all = rall_ref[...], eall_ref[...], gall_ref[...]
    r4, e4 = r4_ref[...], e4_ref[...]

    _H = _B // 2
    zs = [_unpack_bf16(z_ref[0:_H, :]), _unpack_bf16(z_ref[_H:_B, :])]
    xns = [xn_ref[0:_H, :], xn_ref[_H:_B, :]]
    us = [None, None]

    for h in range(2):
        us[h] = _cap_normalize((1.0 / _K) * _fold16(zs[h]) + xns[h], r4, e4)

    for t in range(1, _ROUTIT):
        for h in range(2):
            z, xn, u = zs[h], xns[h], us[h]
            uw = jnp.concatenate([u] * _M, axis=1).astype(jnp.bfloat16)
            p = _dot(z * uw, r_all)
            e = jnp.exp(p)
            s = _dot(e, g_all)
            pb = _dot((e / s).astype(jnp.bfloat16),
                      e_all).astype(jnp.bfloat16)
            u = _fold16(z * pb) + xn
            if t < _ROUTIT - 1:
                u = _cap_normalize(u, r4, e4)
            us[h] = u

    for h in range(2):
        sl = slice(0, _H) if h == 0 else slice(_H, _B)
        if final:
            o_ref[sl, :] = us[h]
            ob_ref[sl, :] = _pack_bf16(us[h])
        else:
            hn = _cap_normalize(jnp.maximum(us[h], 0.0), r4, e4)
            o_ref[sl, :] = hn
            ob_ref[sl, :] = _pack_bf16(hn)


def _route(z2d, xn, final):
    grid = (_N // _B,)
    full = lambda shape: pl.BlockSpec(shape, lambda i: tuple(0 for _ in shape))
    return pl.pallas_call(
        functools.partial(_route_body, final=final),
        grid=grid,
        in_specs=[
            pl.BlockSpec((_B, _W // 2), lambda i: (i, 0)),
            pl.BlockSpec((_B, _D), lambda i: (i, 0)),
            full((_W, _M * _K)),
            full((_M * _K, _W)),
            full((_M * _K, _M * _K)),
            full((_D, _K)),
            full((_K, _D)),
        ],
        out_specs=[
            pl.BlockSpec((_B, _D), lambda i: (i, 0)),
            pl.BlockSpec((_B, _DW), lambda i: (i, 0)),
        ],
        out_shape=[
            jax.ShapeDtypeStruct((_N, _D), jnp.float32),
            jax.ShapeDtypeStruct((_N, _DW), jnp.int32),
        ],
    )(z2d, xn, *_np_consts())


def _head_body(u_ref, awf_ref, attpW_ref, attpb_ref, predW_ref, predb_ref,
               o1_ref, o2_ref, o3_ref, o4_ref):
    u = u_ref[...]
    r4 = _reduce_mat(_NH, _D)
    e4 = _expand_mat(_NH, _D)
    scores = _dot(u * awf_ref[...], r4)
    m = jnp.max(scores, axis=1, keepdims=True)
    e = jnp.exp(scores - m)
    att = e / jnp.sum(e, axis=1, keepdims=True)
    h_att = u * _dot(att, e4)

    ar = _dot(att, attpW_ref[...]) + attpb_ref[...]
    arm = jnp.max(ar, axis=1, keepdims=True)
    ars = ar - arm
    o3 = ars - jnp.log(jnp.sum(jnp.exp(ars), axis=1, keepdims=True))

    logits = _dot(h_att, predW_ref[...]) + predb_ref[...]
    lm = jnp.max(logits, axis=1, keepdims=True)
    ls = logits - lm
    o1 = ls - jnp.log(jnp.sum(jnp.exp(ls), axis=1, keepdims=True))

    o1_ref[...] = o1
    o2_ref[...] = att
    o3_ref[...] = o3
    o4_ref[...] = h_att


def _head(u, att_w, attp_W, attp_b, pred_W, pred_b):
    nclass = pred_W.shape[1]
    awf = att_w.reshape(1, _D)
    predWt = jnp.concatenate([pred_W] * _K, axis=0)
    grid = (_N // _B,)
    full = lambda shape: pl.BlockSpec(shape, lambda i: tuple(0 for _ in shape))
    return pl.pallas_call(
        _head_body,
        grid=grid,
        in_specs=[
            pl.BlockSpec((_B, _D), lambda i: (i, 0)),
            full((1, _D)),
            full((_K, nclass)),
            full((1, nclass)),
            full((_D, nclass)),
            full((1, nclass)),
        ],
        out_specs=[
            pl.BlockSpec((_B, nclass), lambda i: (i, 0)),
            pl.BlockSpec((_B, _K), lambda i: (i, 0)),
            pl.BlockSpec((_B, nclass), lambda i: (i, 0)),
            pl.BlockSpec((_B, _D), lambda i: (i, 0)),
        ],
        out_shape=[
            jax.ShapeDtypeStruct((_N, nclass), jnp.float32),
            jax.ShapeDtypeStruct((_N, _K), jnp.float32),
            jax.ShapeDtypeStruct((_N, nclass), jnp.float32),
            jax.ShapeDtypeStruct((_N, _D), jnp.float32),
        ],
    )(u, awf, attp_W, attp_b.reshape(1, nclass), predWt,
      pred_b.reshape(1, nclass))


def kernel(x, nb, pca_W, pca_b, att_w, pred_W, pred_b, attp_W, attp_b):
    n, m = nb.shape
    idx_pad = jnp.concatenate(
        [nb.reshape(-1), jnp.zeros((_RPAD - n * m,), dtype=nb.dtype)])

    hn, hn_pk = _pca(x, pca_W, pca_b)
    for stage in range(3):
        zi = _gather_sc(hn_pk, idx_pad)
        z2d = zi.reshape(_RPAD * _DW * 2 // _W, _W // 2)
        hn, hn_pk = _route(z2d, hn, final=(stage == 2))
    u = hn

    o1, att, o3, h_att = _head(u, att_w, attp_W, attp_b, pred_W, pred_b)
    return (o1, att, o3, h_att, u)
